# Initial kernel scaffold; baseline (speedup 1.0000x reference)
import jax
import jax.numpy as jnp
from jax.experimental import pallas as pl


def kernel(H, edge_index, frag_mask, W):
    raise NotImplementedError("write your pallas kernel here")


# speedup vs baseline: 19.2277x; 19.2277x over previous
import functools

import jax
import jax.numpy as jnp
from jax import lax
from jax.experimental import pallas as pl
from jax.experimental.pallas import tpu as pltpu
from jax.experimental.pallas import tpu_sc as plsc

N = 10000
E = 320000
D = 128

NC = 2
NS = 16
NW = NC * NS

EPT = E // NW
C = 80
NCHUNK = EPT // C

RCHUNK = 40
NROWCH = N // RCHUNK


def _mask_body(h_ref, f_ref, hm_ref):
    hm_ref[...] = h_ref[...] * f_ref[...]


def _masked_rows(H, frag_f32):
    grid = 5
    blk = N // grid
    return pl.pallas_call(
        _mask_body,
        grid=(grid,),
        in_specs=[
            pl.BlockSpec((blk, D), lambda i: (i, 0)),
            pl.BlockSpec((blk, 1), lambda i: (i, 0)),
        ],
        out_specs=pl.BlockSpec((blk, D), lambda i: (i, 0)),
        out_shape=jax.ShapeDtypeStruct((N, D), jnp.float32),
    )(H, frag_f32)


def _sc_body(hm_hbm, src_hbm, dst_hbm, out_hbm,
             acc, sidx, didx, rows, rbuf, gsem):
    cid = lax.axis_index("c")
    sid = lax.axis_index("s")
    wid = cid * NS + sid
    base = wid * EPT

    def _zero_vec(i, _):
        r = i // (D // 16)
        c = i % (D // 16)
        rbuf[r, pl.ds(c * 16, 16)] = jnp.zeros((16,), jnp.float32)
        return _
    lax.fori_loop(0, RCHUNK * (D // 16), _zero_vec, None)

    def _zero_chunk(j, _):
        k = sid + j * NS
        @pl.when(k < NROWCH)
        def _():
            pltpu.sync_copy(rbuf, acc.at[pl.ds(k * RCHUNK, RCHUNK), :])
        return _
    lax.fori_loop(0, (NROWCH + NS - 1) // NS, _zero_chunk, None)
    plsc.subcore_barrier()

    def _chunk(g, _):
        off = base + g * C
        pltpu.sync_copy(src_hbm.at[pl.ds(off, C)], sidx)
        pltpu.sync_copy(dst_hbm.at[pl.ds(off, C)], didx)
        pltpu.async_copy(hm_hbm.at[sidx], rows, gsem).wait()
        pltpu.sync_copy(rows, acc.at[didx], add=True)
        return _
    lax.fori_loop(0, NCHUNK, _chunk, None)
    plsc.subcore_barrier()

    def _dump_chunk(j, _):
        k = sid + j * NS
        @pl.when(k < NROWCH)
        def _():
            rr = k * RCHUNK
            pltpu.sync_copy(acc.at[pl.ds(rr, RCHUNK), :], rbuf)
            pltpu.sync_copy(rbuf, out_hbm.at[cid, pl.ds(rr, RCHUNK), :])
        return _
    lax.fori_loop(0, (NROWCH + NS - 1) // NS, _dump_chunk, None)


def _sc_scatter(Hm, src_i32, dst_i32):
    mesh = plsc.VectorSubcoreMesh(core_axis_name="c", subcore_axis_name="s")
    f = functools.partial(
        pl.kernel,
        out_type=jax.ShapeDtypeStruct((NC, N, D), jnp.float32),
        mesh=mesh,
        scratch_types=[
            pltpu.VMEM_SHARED((N, D), jnp.float32),
            pltpu.VMEM((C,), jnp.int32),
            pltpu.VMEM((C,), jnp.int32),
            pltpu.VMEM((C, D), jnp.float32),
            pltpu.VMEM((RCHUNK, D), jnp.float32),
            pltpu.SemaphoreType.DMA,
        ],
    )(_sc_body)
    return f(Hm, src_i32, dst_i32)


def _finish_body(h_ref, f_ref, p0_ref, p1_ref, w_ref, out_ref):
    agg = p0_ref[...] + p1_ref[...]
    y = lax.dot_general(agg, w_ref[...], (((1,), (1,)), ((), ())),
                        preferred_element_type=jnp.float32)
    out_ref[...] = h_ref[...] + f_ref[...] * y


def _finish(H, frag_f32, P0, P1, W):
    grid = 5
    blk = N // grid
    return pl.pallas_call(
        _finish_body,
        grid=(grid,),
        in_specs=[
            pl.BlockSpec((blk, D), lambda i: (i, 0)),
            pl.BlockSpec((blk, 1), lambda i: (i, 0)),
            pl.BlockSpec((blk, D), lambda i: (i, 0)),
            pl.BlockSpec((blk, D), lambda i: (i, 0)),
            pl.BlockSpec((D, D), lambda i: (0, 0)),
        ],
        out_specs=pl.BlockSpec((blk, D), lambda i: (i, 0)),
        out_shape=jax.ShapeDtypeStruct((N, D), jnp.float32),
    )(H, frag_f32, P0, P1, W)


def kernel(H, edge_index, frag_mask, W):
    frag_f32 = frag_mask.astype(jnp.float32).reshape(N, 1)
    ei = edge_index.astype(jnp.int32)
    src, dst = ei[0], ei[1]
    Hm = _masked_rows(H, frag_f32)
    P = _sc_scatter(Hm, src, dst)
    return _finish(H, frag_f32, P[0], P[1], W)


# speedup vs baseline: 37.0210x; 1.9254x over previous
import functools

import jax
import jax.numpy as jnp
from jax import lax
from jax.experimental import pallas as pl
from jax.experimental.pallas import tpu as pltpu
from jax.experimental.pallas import tpu_sc as plsc

N = 10000
E = 320000
D = 128

NC = 2
NS = 16
NW = NC * NS

EPT = E // NW
C = 80
NCHUNK = EPT // C

RCHUNK = 40
NROWCH = N // RCHUNK

NBUF = 4
NOUT = NCHUNK // NBUF


def _mask_body(h_ref, f_ref, hm_ref):
    hm_ref[...] = h_ref[...] * f_ref[...]


def _masked_rows(H, frag_f32):
    grid = 5
    blk = N // grid
    return pl.pallas_call(
        _mask_body,
        grid=(grid,),
        in_specs=[
            pl.BlockSpec((blk, D), lambda i: (i, 0)),
            pl.BlockSpec((blk, 1), lambda i: (i, 0)),
        ],
        out_specs=pl.BlockSpec((blk, D), lambda i: (i, 0)),
        out_shape=jax.ShapeDtypeStruct((N, D), jnp.float32),
    )(H, frag_f32)


def _sc_body(hm_hbm, epk_hbm, out_hbm,
             acc, r0, r1, r2, r3, p0, p1, p2, p3, rbuf, isem, gsem, ssem):
    rows = [r0, r1, r2, r3]
    pidx = [p0, p1, p2, p3]
    cid = lax.axis_index("c")
    sid = lax.axis_index("s")
    wid = cid * NS + sid
    base = wid * EPT

    def _zero_vec(i, _):
        r = i // (D // 16)
        c = i % (D // 16)
        rbuf[r, pl.ds(c * 16, 16)] = jnp.zeros((16,), jnp.float32)
        return _
    lax.fori_loop(0, RCHUNK * (D // 16), _zero_vec, None)

    def _zero_chunk(j, _):
        k = sid + j * NS
        @pl.when(k < NROWCH)
        def _():
            pltpu.sync_copy(rbuf, acc.at[pl.ds(k * RCHUNK, RCHUNK), :])
        return _
    lax.fori_loop(0, (NROWCH + NS - 1) // NS, _zero_chunk, None)
    plsc.subcore_barrier()

    def _fire_idx(g, b):
        pltpu.async_copy(epk_hbm.at[wid, g], pidx[b], isem.at[b])

    def _wait_idx(b):
        pltpu.make_async_copy(epk_hbm.at[wid, 0], pidx[b], isem.at[b]).wait()

    def _fire_gather(b):
        pltpu.async_copy(hm_hbm.at[pidx[b].at[0]], rows[b], gsem.at[b])

    def _wait_gather(b):
        pltpu.make_async_copy(
            hm_hbm.at[pidx[b].at[0]], rows[b], gsem.at[b]).wait()

    def _fire_scatter(b):
        pltpu.async_copy(rows[b], acc.at[pidx[b].at[1]], ssem.at[b],
                         add=True)

    def _wait_scatter(b):
        pltpu.make_async_copy(
            rows[b], acc.at[pidx[b].at[1]], ssem.at[b]).wait()

    for b in range(NBUF):
        _fire_idx(b, b)
    for b in range(NBUF):
        _wait_idx(b)
        _fire_gather(b)

    def _outer(t, _):
        for b in range(NBUF):
            _wait_gather(b)
            _fire_scatter(b)
        for b in range(NBUF):
            g2 = (t + 1) * NBUF + b
            @pl.when(g2 < NCHUNK)
            def _():
                _wait_scatter(b)
                _fire_idx(g2, b)
        for b in range(NBUF):
            g2 = (t + 1) * NBUF + b
            @pl.when(g2 < NCHUNK)
            def _():
                _wait_idx(b)
                _fire_gather(b)
        return _
    lax.fori_loop(0, NOUT, _outer, None)

    for b in range(NCHUNK - NOUT * NBUF):
        _wait_gather(b)
        _fire_scatter(b)
    for b in range(NBUF):
        _wait_scatter(b)
    plsc.subcore_barrier()

    def _dump_chunk(j, _):
        k = sid + j * NS
        @pl.when(k < NROWCH)
        def _():
            rr = k * RCHUNK
            pltpu.sync_copy(acc.at[pl.ds(rr, RCHUNK), :], rbuf)
            pltpu.sync_copy(rbuf, out_hbm.at[cid, pl.ds(rr, RCHUNK), :])
        return _
    lax.fori_loop(0, (NROWCH + NS - 1) // NS, _dump_chunk, None)


def _sc_scatter(Hm, edge_packed):
    mesh = plsc.VectorSubcoreMesh(core_axis_name="c", subcore_axis_name="s")
    f = functools.partial(
        pl.kernel,
        out_type=jax.ShapeDtypeStruct((NC, N, D), jnp.float32),
        mesh=mesh,
        scratch_types=[
            pltpu.VMEM_SHARED((N, D), jnp.float32),
        ] + [pltpu.VMEM((C, D), jnp.float32)] * NBUF + [
        ] + [pltpu.VMEM((2, C), jnp.int32)] * NBUF + [
            pltpu.VMEM((RCHUNK, D), jnp.float32),
            pltpu.SemaphoreType.DMA((NBUF,)),
            pltpu.SemaphoreType.DMA((NBUF,)),
            pltpu.SemaphoreType.DMA((NBUF,)),
        ],
    )(_sc_body)
    return f(Hm, edge_packed)


def _finish_body(h_ref, f_ref, p0_ref, p1_ref, w_ref, out_ref):
    agg = p0_ref[...] + p1_ref[...]
    y = lax.dot_general(agg, w_ref[...], (((1,), (1,)), ((), ())),
                        preferred_element_type=jnp.float32)
    out_ref[...] = h_ref[...] + f_ref[...] * y


def _finish(H, frag_f32, P0, P1, W):
    grid = 5
    blk = N // grid
    return pl.pallas_call(
        _finish_body,
        grid=(grid,),
        in_specs=[
            pl.BlockSpec((blk, D), lambda i: (i, 0)),
            pl.BlockSpec((blk, 1), lambda i: (i, 0)),
            pl.BlockSpec((blk, D), lambda i: (i, 0)),
            pl.BlockSpec((blk, D), lambda i: (i, 0)),
            pl.BlockSpec((D, D), lambda i: (0, 0)),
        ],
        out_specs=pl.BlockSpec((blk, D), lambda i: (i, 0)),
        out_shape=jax.ShapeDtypeStruct((N, D), jnp.float32),
    )(H, frag_f32, P0, P1, W)


def kernel(H, edge_index, frag_mask, W):
    frag_f32 = frag_mask.astype(jnp.float32).reshape(N, 1)
    ei = edge_index.astype(jnp.int32)
    edge_packed = ei.reshape(2, NW, NCHUNK, C).transpose(1, 2, 0, 3)
    Hm = _masked_rows(H, frag_f32)
    P = _sc_scatter(Hm, edge_packed)
    return _finish(H, frag_f32, P[0], P[1], W)


# speedup vs baseline: 39.5699x; 1.0689x over previous
import functools

import jax
import jax.numpy as jnp
from jax import lax
from jax.experimental import pallas as pl
from jax.experimental.pallas import tpu as pltpu
from jax.experimental.pallas import tpu_sc as plsc

N = 10000
E = 320000
D = 128

NC = 2
NS = 16
NW = NC * NS

EPT = E // NW
C = 80
NCHUNK = EPT // C

RCHUNK = 40
NROWCH = N // RCHUNK

NBUF = 4
NOUT = NCHUNK // NBUF


def _mask_body(h_ref, f_ref, hm_ref):
    hm_ref[...] = h_ref[...] * f_ref[...]


def _masked_rows(H, frag_f32):
    grid = 5
    blk = N // grid
    return pl.pallas_call(
        _mask_body,
        grid=(grid,),
        in_specs=[
            pl.BlockSpec((blk, D), lambda i: (i, 0)),
            pl.BlockSpec((blk, 1), lambda i: (i, 0)),
        ],
        out_specs=pl.BlockSpec((blk, D), lambda i: (i, 0)),
        out_shape=jax.ShapeDtypeStruct((N, D), jnp.float32),
    )(H, frag_f32)


def _sc_body(hm_hbm, src_hbm, dst_hbm, out_hbm,
             acc, r0, r1, r2, r3, s0, s1, s2, s3, d0, d1, d2, d3,
             rbuf, isem, gsem, ssem):
    rows = [r0, r1, r2, r3]
    sidx = [s0, s1, s2, s3]
    didx = [d0, d1, d2, d3]
    cid = lax.axis_index("c")
    sid = lax.axis_index("s")
    wid = cid * NS + sid
    base = wid * EPT

    def _zero_vec(i, _):
        r = i // (D // 16)
        c = i % (D // 16)
        rbuf[r, pl.ds(c * 16, 16)] = jnp.zeros((16,), jnp.float32)
        return _
    lax.fori_loop(0, RCHUNK * (D // 16), _zero_vec, None)

    def _zero_chunk(j, _):
        k = sid + j * NS
        @pl.when(k < NROWCH)
        def _():
            pltpu.sync_copy(rbuf, acc.at[pl.ds(k * RCHUNK, RCHUNK), :])
        return _
    lax.fori_loop(0, (NROWCH + NS - 1) // NS, _zero_chunk, None)
    plsc.subcore_barrier()

    def _fire_idx(g, b):
        off = base + g * C
        pltpu.async_copy(src_hbm.at[pl.ds(off, C)], sidx[b], isem.at[b])
        pltpu.async_copy(dst_hbm.at[pl.ds(off, C)], didx[b], isem.at[b])

    def _wait_idx(b):
        pltpu.make_async_copy(
            src_hbm.at[pl.ds(0, C)], sidx[b], isem.at[b]).wait()
        pltpu.make_async_copy(
            dst_hbm.at[pl.ds(0, C)], didx[b], isem.at[b]).wait()

    def _fire_gather(b):
        pltpu.async_copy(hm_hbm.at[sidx[b]], rows[b], gsem.at[b])

    def _wait_gather(b):
        pltpu.make_async_copy(
            hm_hbm.at[sidx[b]], rows[b], gsem.at[b]).wait()

    def _fire_scatter(b):
        pltpu.async_copy(rows[b], acc.at[didx[b]], ssem.at[b], add=True)

    def _wait_scatter(b):
        pltpu.make_async_copy(
            rows[b], acc.at[didx[b]], ssem.at[b]).wait()

    for b in range(NBUF):
        _fire_idx(b, b)
    for b in range(NBUF):
        _wait_idx(b)
        _fire_gather(b)

    def _outer(t, _):
        for b in range(NBUF):
            _wait_gather(b)
            _fire_scatter(b)
        for b in range(NBUF):
            g2 = (t + 1) * NBUF + b
            @pl.when(g2 < NCHUNK)
            def _():
                _wait_scatter(b)
                _fire_idx(g2, b)
                _wait_idx(b)
                _fire_gather(b)
        return _
    lax.fori_loop(0, NOUT, _outer, None)

    for b in range(NCHUNK - NOUT * NBUF):
        _wait_gather(b)
        _fire_scatter(b)
    for b in range(NBUF):
        _wait_scatter(b)
    plsc.subcore_barrier()

    def _dump_chunk(j, _):
        k = sid + j * NS
        @pl.when(k < NROWCH)
        def _():
            rr = k * RCHUNK
            pltpu.sync_copy(acc.at[pl.ds(rr, RCHUNK), :], rbuf)
            pltpu.sync_copy(rbuf, out_hbm.at[cid, pl.ds(rr, RCHUNK), :])
        return _
    lax.fori_loop(0, (NROWCH + NS - 1) // NS, _dump_chunk, None)


def _sc_scatter(Hm, src_i32, dst_i32):
    mesh = plsc.VectorSubcoreMesh(core_axis_name="c", subcore_axis_name="s")
    f = functools.partial(
        pl.kernel,
        out_type=jax.ShapeDtypeStruct((NC, N, D), jnp.float32),
        mesh=mesh,
        scratch_types=[
            pltpu.VMEM_SHARED((N, D), jnp.float32),
        ] + [pltpu.VMEM((C, D), jnp.float32)] * NBUF + [
        ] + [pltpu.VMEM((C,), jnp.int32)] * NBUF + [
        ] + [pltpu.VMEM((C,), jnp.int32)] * NBUF + [
            pltpu.VMEM((RCHUNK, D), jnp.float32),
            pltpu.SemaphoreType.DMA((NBUF,)),
            pltpu.SemaphoreType.DMA((NBUF,)),
            pltpu.SemaphoreType.DMA((NBUF,)),
        ],
    )(_sc_body)
    return f(Hm, src_i32, dst_i32)


def _finish_body(h_ref, f_ref, p0_ref, p1_ref, w_ref, out_ref):
    agg = p0_ref[...] + p1_ref[...]
    y = lax.dot_general(agg, w_ref[...], (((1,), (1,)), ((), ())),
                        preferred_element_type=jnp.float32)
    out_ref[...] = h_ref[...] + f_ref[...] * y


def _finish(H, frag_f32, P0, P1, W):
    grid = 5
    blk = N // grid
    return pl.pallas_call(
        _finish_body,
        grid=(grid,),
        in_specs=[
            pl.BlockSpec((blk, D), lambda i: (i, 0)),
            pl.BlockSpec((blk, 1), lambda i: (i, 0)),
            pl.BlockSpec((blk, D), lambda i: (i, 0)),
            pl.BlockSpec((blk, D), lambda i: (i, 0)),
            pl.BlockSpec((D, D), lambda i: (0, 0)),
        ],
        out_specs=pl.BlockSpec((blk, D), lambda i: (i, 0)),
        out_shape=jax.ShapeDtypeStruct((N, D), jnp.float32),
    )(H, frag_f32, P0, P1, W)


def kernel(H, edge_index, frag_mask, W):
    frag_f32 = frag_mask.astype(jnp.float32).reshape(N, 1)
    ei = edge_index.astype(jnp.int32)
    Hm = _masked_rows(H, frag_f32)
    P = _sc_scatter(Hm, ei[0], ei[1])
    return _finish(H, frag_f32, P[0], P[1], W)


# speedup vs baseline: 40.6879x; 1.0283x over previous
import functools

import jax
import jax.numpy as jnp
from jax import lax
from jax.experimental import pallas as pl
from jax.experimental.pallas import tpu as pltpu
from jax.experimental.pallas import tpu_sc as plsc

N = 10000
E = 320000
D = 128

NC = 2
NS = 16
NW = NC * NS

EPT = E // NW
C = 40
NCHUNK = EPT // C

RCHUNK = 40
NROWCH = N // RCHUNK

NBUF = 5
NOUT = NCHUNK // NBUF


def _mask_body(h_ref, f_ref, hm_ref):
    hm_ref[...] = h_ref[...] * f_ref[...]


def _masked_rows(H, frag_f32):
    grid = 5
    blk = N // grid
    return pl.pallas_call(
        _mask_body,
        grid=(grid,),
        in_specs=[
            pl.BlockSpec((blk, D), lambda i: (i, 0)),
            pl.BlockSpec((blk, 1), lambda i: (i, 0)),
        ],
        out_specs=pl.BlockSpec((blk, D), lambda i: (i, 0)),
        out_shape=jax.ShapeDtypeStruct((N, D), jnp.float32),
    )(H, frag_f32)


def _sc_body(hm_hbm, src_hbm, dst_hbm, out_hbm,
             acc, r0, r1, r2, r3, r4, d0, d1, d2, d3, d4, sidx_all,
             isem, gsem, ssem):
    rows = [r0, r1, r2, r3, r4]
    didx = [d0, d1, d2, d3, d4]
    rbuf = rows[0]
    cid = lax.axis_index("c")
    sid = lax.axis_index("s")
    wid = cid * NS + sid
    base = wid * EPT

    def _zero_vec(i, _):
        r = i // (D // 16)
        c = i % (D // 16)
        rbuf[r, pl.ds(c * 16, 16)] = jnp.zeros((16,), jnp.float32)
        return _
    lax.fori_loop(0, RCHUNK * (D // 16), _zero_vec, None)

    def _zero_chunk(j, _):
        k = sid + j * NS
        @pl.when(k < NROWCH)
        def _():
            pltpu.sync_copy(rbuf, acc.at[pl.ds(k * RCHUNK, RCHUNK), :])
        return _
    lax.fori_loop(0, (NROWCH + NS - 1) // NS, _zero_chunk, None)

    pltpu.sync_copy(src_hbm.at[pl.ds(base, EPT)], sidx_all)
    plsc.subcore_barrier()

    def _fire_didx(g, b):
        pltpu.async_copy(dst_hbm.at[pl.ds(base + g * C, C)], didx[b],
                         isem.at[b])

    def _wait_didx(b):
        pltpu.make_async_copy(
            dst_hbm.at[pl.ds(0, C)], didx[b], isem.at[b]).wait()

    def _fire_gather(g, b):
        pltpu.async_copy(
            hm_hbm.at[sidx_all.at[pl.ds(g * C, C)]], rows[b], gsem.at[b])

    def _wait_gather(g, b):
        pltpu.make_async_copy(
            hm_hbm.at[sidx_all.at[pl.ds(g * C, C)]], rows[b],
            gsem.at[b]).wait()

    def _fire_scatter(b):
        pltpu.async_copy(rows[b], acc.at[didx[b]], ssem.at[b], add=True)

    def _wait_scatter(b):
        pltpu.make_async_copy(
            rows[b], acc.at[didx[b]], ssem.at[b]).wait()

    for b in range(NBUF):
        _fire_didx(b, b)
        _fire_gather(b, b)

    def _outer(t, _):
        for b in range(NBUF):
            _wait_gather(t * NBUF + b, b)
            _wait_didx(b)
            _fire_scatter(b)
        for b in range(NBUF):
            g2 = (t + 1) * NBUF + b
            @pl.when(g2 < NCHUNK)
            def _():
                _wait_scatter(b)
                _fire_didx(g2, b)
                _fire_gather(g2, b)
        return _
    lax.fori_loop(0, NOUT, _outer, None)

    for b in range(NBUF):
        _wait_scatter(b)
    plsc.subcore_barrier()

    def _dump_chunk(j, _):
        k = sid + j * NS
        @pl.when(k < NROWCH)
        def _():
            rr = k * RCHUNK
            pltpu.sync_copy(acc.at[pl.ds(rr, RCHUNK), :], rbuf)
            pltpu.sync_copy(rbuf, out_hbm.at[cid, pl.ds(rr, RCHUNK), :])
        return _
    lax.fori_loop(0, (NROWCH + NS - 1) // NS, _dump_chunk, None)


def _sc_scatter(Hm, src_i32, dst_i32):
    mesh = plsc.VectorSubcoreMesh(core_axis_name="c", subcore_axis_name="s")
    f = functools.partial(
        pl.kernel,
        out_type=jax.ShapeDtypeStruct((NC, N, D), jnp.float32),
        mesh=mesh,
        scratch_types=[
            pltpu.VMEM_SHARED((N, D), jnp.float32),
        ] + [pltpu.VMEM((C, D), jnp.float32)] * NBUF + [
        ] + [pltpu.VMEM((C,), jnp.int32)] * NBUF + [
            pltpu.VMEM((EPT,), jnp.int32),
            pltpu.SemaphoreType.DMA((NBUF,)),
            pltpu.SemaphoreType.DMA((NBUF,)),
            pltpu.SemaphoreType.DMA((NBUF,)),
        ],
    )(_sc_body)
    return f(Hm, src_i32, dst_i32)


def _finish_body(h_ref, f_ref, p0_ref, p1_ref, w_ref, out_ref):
    agg = p0_ref[...] + p1_ref[...]
    y = lax.dot_general(agg, w_ref[...], (((1,), (1,)), ((), ())),
                        preferred_element_type=jnp.float32)
    out_ref[...] = h_ref[...] + f_ref[...] * y


def _finish(H, frag_f32, P0, P1, W):
    grid = 5
    blk = N // grid
    return pl.pallas_call(
        _finish_body,
        grid=(grid,),
        in_specs=[
            pl.BlockSpec((blk, D), lambda i: (i, 0)),
            pl.BlockSpec((blk, 1), lambda i: (i, 0)),
            pl.BlockSpec((blk, D), lambda i: (i, 0)),
            pl.BlockSpec((blk, D), lambda i: (i, 0)),
            pl.BlockSpec((D, D), lambda i: (0, 0)),
        ],
        out_specs=pl.BlockSpec((blk, D), lambda i: (i, 0)),
        out_shape=jax.ShapeDtypeStruct((N, D), jnp.float32),
    )(H, frag_f32, P0, P1, W)


def kernel(H, edge_index, frag_mask, W):
    frag_f32 = frag_mask.astype(jnp.float32).reshape(N, 1)
    ei = edge_index.astype(jnp.int32)
    Hm = _masked_rows(H, frag_f32)
    P = _sc_scatter(Hm, ei[0], ei[1])
    return _finish(H, frag_f32, P[0], P[1], W)


# speedup vs baseline: 46.2758x; 1.1373x over previous
import functools

import jax
import jax.numpy as jnp
from jax import lax
from jax.experimental import pallas as pl
from jax.experimental.pallas import tpu as pltpu
from jax.experimental.pallas import tpu_sc as plsc

N = 10000
E = 320000
D = 128

NC = 2
NS = 16
NW = NC * NS

EPT = E // NW
C = 40
NCHUNK = EPT // C

RCHUNK = 40
NROWCH = N // RCHUNK

NBUF = 5
NOUT = NCHUNK // NBUF


def _mask_body(h_ref, f_ref, e_ref, hm_ref, src_ref, dst_ref):
    hm_ref[...] = h_ref[...] * f_ref[...]
    src_ref[...] = e_ref[0, :]
    dst_ref[...] = e_ref[1, :]


def _masked_rows(H, frag_col, edge_index_i32):
    grid = 5
    blk = N // grid
    eblk = 65536
    return pl.pallas_call(
        _mask_body,
        grid=(grid,),
        in_specs=[
            pl.BlockSpec((blk, D), lambda i: (i, 0)),
            pl.BlockSpec((blk, 1), lambda i: (i, 0)),
            pl.BlockSpec((2, eblk), lambda i: (0, i)),
        ],
        out_specs=[
            pl.BlockSpec((blk, D), lambda i: (i, 0)),
            pl.BlockSpec((eblk,), lambda i: (i,)),
            pl.BlockSpec((eblk,), lambda i: (i,)),
        ],
        out_shape=[
            jax.ShapeDtypeStruct((N, D), jnp.float32),
            jax.ShapeDtypeStruct((E,), jnp.int32),
            jax.ShapeDtypeStruct((E,), jnp.int32),
        ],
    )(H, frag_col, edge_index_i32)


def _sc_body(hm_hbm, src_hbm, dst_hbm, out_hbm,
             acc, r0, r1, r2, r3, r4, d0, d1, d2, d3, d4, sidx_all,
             isem, gsem, ssem):
    rows = [r0, r1, r2, r3, r4]
    didx = [d0, d1, d2, d3, d4]
    rbuf = rows[0]
    cid = lax.axis_index("c")
    sid = lax.axis_index("s")
    wid = cid * NS + sid
    base = wid * EPT

    def _zero_vec(i, _):
        r = i // (D // 16)
        c = i % (D // 16)
        rbuf[r, pl.ds(c * 16, 16)] = jnp.zeros((16,), jnp.float32)
        return _
    lax.fori_loop(0, RCHUNK * (D // 16), _zero_vec, None)

    def _zero_chunk(j, _):
        k = sid + j * NS
        @pl.when(k < NROWCH)
        def _():
            pltpu.sync_copy(rbuf, acc.at[pl.ds(k * RCHUNK, RCHUNK), :])
        return _
    lax.fori_loop(0, (NROWCH + NS - 1) // NS, _zero_chunk, None)

    pltpu.sync_copy(src_hbm.at[pl.ds(base, EPT)], sidx_all)
    plsc.subcore_barrier()

    def _fire_didx(g, b):
        pltpu.async_copy(dst_hbm.at[pl.ds(base + g * C, C)], didx[b],
                         isem.at[b])

    def _wait_didx(b):
        pltpu.make_async_copy(
            dst_hbm.at[pl.ds(0, C)], didx[b], isem.at[b]).wait()

    def _fire_gather(g, b):
        pltpu.async_copy(
            hm_hbm.at[sidx_all.at[pl.ds(g * C, C)]], rows[b], gsem.at[b])

    def _wait_gather(g, b):
        pltpu.make_async_copy(
            hm_hbm.at[sidx_all.at[pl.ds(g * C, C)]], rows[b],
            gsem.at[b]).wait()

    def _fire_scatter(b):
        pltpu.async_copy(rows[b], acc.at[didx[b]], ssem.at[b], add=True)

    def _wait_scatter(b):
        pltpu.make_async_copy(
            rows[b], acc.at[didx[b]], ssem.at[b]).wait()

    for b in range(NBUF):
        _fire_didx(b, b)
        _fire_gather(b, b)

    def _outer(t, _):
        for b in range(NBUF):
            _wait_gather(t * NBUF + b, b)
            _wait_didx(b)
            _fire_scatter(b)
        for b in range(NBUF):
            g2 = (t + 1) * NBUF + b
            @pl.when(g2 < NCHUNK)
            def _():
                _wait_scatter(b)
                _fire_didx(g2, b)
                _fire_gather(g2, b)
        return _
    lax.fori_loop(0, NOUT, _outer, None)

    for b in range(NBUF):
        _wait_scatter(b)
    plsc.subcore_barrier()

    def _dump_chunk(j, _):
        k = sid + j * NS
        @pl.when(k < NROWCH)
        def _():
            rr = k * RCHUNK
            pltpu.sync_copy(acc.at[pl.ds(rr, RCHUNK), :], rbuf)
            pltpu.sync_copy(rbuf, out_hbm.at[cid, pl.ds(rr, RCHUNK), :])
        return _
    lax.fori_loop(0, (NROWCH + NS - 1) // NS, _dump_chunk, None)


def _sc_scatter(Hm, src_i32, dst_i32):
    mesh = plsc.VectorSubcoreMesh(core_axis_name="c", subcore_axis_name="s")
    f = functools.partial(
        pl.kernel,
        out_type=jax.ShapeDtypeStruct((NC, N, D), jnp.float32),
        mesh=mesh,
        scratch_types=[
            pltpu.VMEM_SHARED((N, D), jnp.float32),
        ] + [pltpu.VMEM((C, D), jnp.float32)] * NBUF + [
        ] + [pltpu.VMEM((C,), jnp.int32)] * NBUF + [
            pltpu.VMEM((EPT,), jnp.int32),
            pltpu.SemaphoreType.DMA((NBUF,)),
            pltpu.SemaphoreType.DMA((NBUF,)),
            pltpu.SemaphoreType.DMA((NBUF,)),
        ],
    )(_sc_body)
    return f(Hm, src_i32, dst_i32)


def _finish_body(h_ref, f_ref, p0_ref, p1_ref, w_ref, out_ref):
    agg = p0_ref[0] + p1_ref[0]
    y = lax.dot_general(agg, w_ref[...], (((1,), (1,)), ((), ())),
                        preferred_element_type=jnp.float32)
    out_ref[...] = h_ref[...] + f_ref[...] * y


def _finish(H, frag_col, P, W):
    grid = 5
    blk = N // grid
    return pl.pallas_call(
        _finish_body,
        grid=(grid,),
        in_specs=[
            pl.BlockSpec((blk, D), lambda i: (i, 0)),
            pl.BlockSpec((blk, 1), lambda i: (i, 0)),
            pl.BlockSpec((1, blk, D), lambda i: (0, i, 0)),
            pl.BlockSpec((1, blk, D), lambda i: (1, i, 0)),
            pl.BlockSpec((D, D), lambda i: (0, 0)),
        ],
        out_specs=pl.BlockSpec((blk, D), lambda i: (i, 0)),
        out_shape=jax.ShapeDtypeStruct((N, D), jnp.float32),
    )(H, frag_col, P, P, W)


def kernel(H, edge_index, frag_mask, W):
    frag_col = frag_mask.reshape(N, 1).astype(jnp.float32)
    ei = edge_index.astype(jnp.int32)
    Hm, src, dst = _masked_rows(H, frag_col, ei)
    P = _sc_scatter(Hm, src, dst)
    return _finish(H, frag_col, P, W)


# speedup vs baseline: 47.0829x; 1.0174x over previous
import functools

import jax
import jax.numpy as jnp
from jax import lax
from jax.experimental import pallas as pl
from jax.experimental.pallas import tpu as pltpu
from jax.experimental.pallas import tpu_sc as plsc

N = 10000
E = 320000
D = 128

NC = 2
NS = 16
NW = NC * NS

EPT = E // NW
C = 40
NCHUNK = EPT // C

RCHUNK = 40
NROWCH = N // RCHUNK

NBUF = 5
NOUT = NCHUNK // NBUF


def _mask_body(h_ref, f_ref, e_ref, hm_ref, src_ref, dst_ref):
    hm_ref[...] = h_ref[...] * f_ref[...]
    src_ref[...] = e_ref[0, :]
    dst_ref[...] = e_ref[1, :]


def _masked_rows(H, frag_col, edge_index_i32):
    grid = 5
    blk = N // grid
    eblk = 65536
    return pl.pallas_call(
        _mask_body,
        grid=(grid,),
        in_specs=[
            pl.BlockSpec((blk, D), lambda i: (i, 0)),
            pl.BlockSpec((blk, 1), lambda i: (i, 0)),
            pl.BlockSpec((2, eblk), lambda i: (0, i)),
        ],
        out_specs=[
            pl.BlockSpec((blk, D), lambda i: (i, 0)),
            pl.BlockSpec((eblk,), lambda i: (i,)),
            pl.BlockSpec((eblk,), lambda i: (i,)),
        ],
        out_shape=[
            jax.ShapeDtypeStruct((N, D), jnp.float32),
            jax.ShapeDtypeStruct((E,), jnp.int32),
            jax.ShapeDtypeStruct((E,), jnp.int32),
        ],
    )(H, frag_col, edge_index_i32)


def _sc_body(hm_hbm, src_hbm, dst_hbm, out_hbm,
             acc, r0, r1, r2, r3, r4, d0, d1, d2, d3, d4, sidx_all,
             isem, gsem, ssem):
    rows = [r0, r1, r2, r3, r4]
    didx = [d0, d1, d2, d3, d4]
    rbuf = rows[0]
    cid = lax.axis_index("c")
    sid = lax.axis_index("s")
    wid = cid * NS + sid
    base = wid * EPT

    def _zero_vec(i, _):
        r = i // (D // 16)
        c = i % (D // 16)
        rbuf[r, pl.ds(c * 16, 16)] = jnp.zeros((16,), jnp.float32)
        return _
    lax.fori_loop(0, RCHUNK * (D // 16), _zero_vec, None)

    def _zero_fire(j, _):
        k = sid + j * NS
        @pl.when(k < NROWCH)
        def _():
            pltpu.async_copy(rbuf, acc.at[pl.ds(k * RCHUNK, RCHUNK), :],
                             gsem.at[0])
        return _
    lax.fori_loop(0, (NROWCH + NS - 1) // NS, _zero_fire, None)

    def _zero_drain(j, _):
        k = sid + j * NS
        @pl.when(k < NROWCH)
        def _():
            pltpu.make_async_copy(
                rbuf, acc.at[pl.ds(k * RCHUNK, RCHUNK), :],
                gsem.at[0]).wait()
        return _
    lax.fori_loop(0, (NROWCH + NS - 1) // NS, _zero_drain, None)

    pltpu.sync_copy(src_hbm.at[pl.ds(base, EPT)], sidx_all)
    plsc.subcore_barrier()

    def _fire_didx(g, b):
        pltpu.async_copy(dst_hbm.at[pl.ds(base + g * C, C)], didx[b],
                         isem.at[b])

    def _wait_didx(b):
        pltpu.make_async_copy(
            dst_hbm.at[pl.ds(0, C)], didx[b], isem.at[b]).wait()

    def _fire_gather(g, b):
        pltpu.async_copy(
            hm_hbm.at[sidx_all.at[pl.ds(g * C, C)]], rows[b], gsem.at[b])

    def _wait_gather(g, b):
        pltpu.make_async_copy(
            hm_hbm.at[sidx_all.at[pl.ds(g * C, C)]], rows[b],
            gsem.at[b]).wait()

    def _fire_scatter(b):
        pltpu.async_copy(rows[b], acc.at[didx[b]], ssem.at[b], add=True)

    def _wait_scatter(b):
        pltpu.make_async_copy(
            rows[b], acc.at[didx[b]], ssem.at[b]).wait()

    for b in range(NBUF):
        _fire_didx(b, b)
        _fire_gather(b, b)

    def _outer(t, _):
        for b in range(NBUF):
            _wait_gather(t * NBUF + b, b)
            _wait_didx(b)
            _fire_scatter(b)
        for b in range(NBUF):
            g2 = (t + 1) * NBUF + b
            @pl.when(g2 < NCHUNK)
            def _():
                _wait_scatter(b)
                _fire_didx(g2, b)
                _fire_gather(g2, b)
        return _
    lax.fori_loop(0, NOUT, _outer, None)

    for b in range(NBUF):
        _wait_scatter(b)
    plsc.subcore_barrier()

    def _dump_fire(j, _):
        k = sid + j * NS
        @pl.when(k < NROWCH)
        def _():
            rr = k * RCHUNK
            pltpu.async_copy(acc.at[pl.ds(rr, RCHUNK), :],
                             out_hbm.at[cid, pl.ds(rr, RCHUNK), :],
                             gsem.at[0])
        return _
    lax.fori_loop(0, (NROWCH + NS - 1) // NS, _dump_fire, None)

    def _dump_drain(j, _):
        k = sid + j * NS
        @pl.when(k < NROWCH)
        def _():
            rr = k * RCHUNK
            pltpu.make_async_copy(
                acc.at[pl.ds(rr, RCHUNK), :],
                out_hbm.at[cid, pl.ds(rr, RCHUNK), :], gsem.at[0]).wait()
        return _
    lax.fori_loop(0, (NROWCH + NS - 1) // NS, _dump_drain, None)


def _sc_scatter(Hm, src_i32, dst_i32):
    mesh = plsc.VectorSubcoreMesh(core_axis_name="c", subcore_axis_name="s")
    f = functools.partial(
        pl.kernel,
        out_type=jax.ShapeDtypeStruct((NC, N, D), jnp.float32),
        mesh=mesh,
        scratch_types=[
            pltpu.VMEM_SHARED((N, D), jnp.float32),
        ] + [pltpu.VMEM((C, D), jnp.float32)] * NBUF + [
        ] + [pltpu.VMEM((C,), jnp.int32)] * NBUF + [
            pltpu.VMEM((EPT,), jnp.int32),
            pltpu.SemaphoreType.DMA((NBUF,)),
            pltpu.SemaphoreType.DMA((NBUF,)),
            pltpu.SemaphoreType.DMA((NBUF,)),
        ],
    )(_sc_body)
    return f(Hm, src_i32, dst_i32)


def _finish_body(h_ref, f_ref, p0_ref, p1_ref, w_ref, out_ref):
    agg = p0_ref[0] + p1_ref[0]
    y = lax.dot_general(agg, w_ref[...], (((1,), (1,)), ((), ())),
                        preferred_element_type=jnp.float32)
    out_ref[...] = h_ref[...] + f_ref[...] * y


def _finish(H, frag_col, P, W):
    grid = 5
    blk = N // grid
    return pl.pallas_call(
        _finish_body,
        grid=(grid,),
        in_specs=[
            pl.BlockSpec((blk, D), lambda i: (i, 0)),
            pl.BlockSpec((blk, 1), lambda i: (i, 0)),
            pl.BlockSpec((1, blk, D), lambda i: (0, i, 0)),
            pl.BlockSpec((1, blk, D), lambda i: (1, i, 0)),
            pl.BlockSpec((D, D), lambda i: (0, 0)),
        ],
        out_specs=pl.BlockSpec((blk, D), lambda i: (i, 0)),
        out_shape=jax.ShapeDtypeStruct((N, D), jnp.float32),
    )(H, frag_col, P, P, W)


def kernel(H, edge_index, frag_mask, W):
    frag_col = frag_mask.reshape(N, 1).astype(jnp.float32)
    ei = edge_index.astype(jnp.int32)
    Hm, src, dst = _masked_rows(H, frag_col, ei)
    P = _sc_scatter(Hm, src, dst)
    return _finish(H, frag_col, P, W)
